# Initial kernel scaffold; baseline (speedup 1.0000x reference)
import jax
import jax.numpy as jnp
from jax.experimental import pallas as pl


def kernel(data, edge_index, W1, b1, W2, b2, W3, b3):
    raise NotImplementedError("write your pallas kernel here")


# speedup vs baseline: 13.0640x; 13.0640x over previous
import functools

import jax
import jax.numpy as jnp
from jax import lax
from jax.experimental import pallas as pl
from jax.experimental.pallas import tpu as pltpu
from jax.experimental.pallas import tpu_sc as plsc

N = 10000
E = 320000
D = 128

NC = 2
NS = 16
NP = 10240
TROWS = NP // NS
K = 80
EPW = E // (NC * NS)
CHUNKS = EPW // K

_sc_mesh = plsc.VectorSubcoreMesh(core_axis_name="c", subcore_axis_name="s")


@functools.partial(
    pl.kernel,
    out_type=jax.ShapeDtypeStruct((NC * NP,), jnp.float32),
    mesh=_sc_mesh,
    scratch_types=[
        pltpu.VMEM_SHARED((NP,), jnp.float32),
        pltpu.VMEM((K,), jnp.int32),
        pltpu.VMEM((K,), jnp.float32),
        pltpu.SemaphoreType.DMA,
        pltpu.SemaphoreType.DMA,
    ],
)
def _sc_degree(dst_hbm, zeros_hbm, ones_hbm, out_hbm, acc, di, ones_v, sem0, sem1):
    c = lax.axis_index("c")
    s = lax.axis_index("s")
    row_slice = pl.ds(s * TROWS, TROWS)
    cp_ones = pltpu.async_copy(ones_hbm, ones_v, sem1)
    pltpu.sync_copy(zeros_hbm, acc.at[row_slice])
    cp_ones.wait()
    plsc.subcore_barrier()

    base = (c * NS + s) * EPW

    @pl.loop(0, CHUNKS)
    def _(i):
        pltpu.async_copy(dst_hbm.at[pl.ds(base + i * K, K)], di, sem0).wait()
        pltpu.sync_copy(ones_v, acc.at[di], add=True)

    plsc.subcore_barrier()
    pltpu.sync_copy(acc.at[row_slice], out_hbm.at[pl.ds(c * NP + s * TROWS, TROWS)])


@functools.partial(
    pl.kernel,
    out_type=jax.ShapeDtypeStruct((NC, NP, D), jnp.float32),
    mesh=_sc_mesh,
    scratch_types=[
        pltpu.VMEM_SHARED((NP, D), jnp.float32),
        pltpu.VMEM((K,), jnp.int32),
        pltpu.VMEM((K,), jnp.int32),
        pltpu.VMEM((K, D), jnp.float32),
        pltpu.SemaphoreType.DMA,
        pltpu.SemaphoreType.DMA,
        pltpu.SemaphoreType.DMA,
    ],
)
def _sc_scatter(y_hbm, src_hbm, dst_hbm, zeros_hbm, out_hbm,
                acc, si, di, rows, sem0, sem1, sem2):
    c = lax.axis_index("c")
    s = lax.axis_index("s")
    row_slice = pl.ds(s * TROWS, TROWS)

    @pl.when(c == 0)
    def _():
        pltpu.sync_copy(y_hbm.at[row_slice, :], acc.at[row_slice, :])

    @pl.when(c != 0)
    def _():
        pltpu.sync_copy(zeros_hbm, acc.at[row_slice, :])

    plsc.subcore_barrier()

    base = (c * NS + s) * EPW

    @pl.loop(0, CHUNKS)
    def _(i):
        cp_s = pltpu.async_copy(src_hbm.at[pl.ds(base + i * K, K)], si, sem0)
        cp_d = pltpu.async_copy(dst_hbm.at[pl.ds(base + i * K, K)], di, sem1)
        cp_s.wait()
        pltpu.async_copy(y_hbm.at[si], rows, sem2).wait()
        cp_d.wait()
        pltpu.sync_copy(rows, acc.at[di], add=True)

    plsc.subcore_barrier()
    pltpu.sync_copy(acc.at[row_slice, :], out_hbm.at[c, row_slice, :])


RB = 1024
GRID = NP // RB
FRB = 1000
FGRID = N // FRB

_HI = jax.lax.Precision.HIGHEST


def _mm_body(x_ref, w_ref, o_ref):
    o_ref[...] = jnp.dot(x_ref[...], w_ref[...], precision=_HI)


_tc_matmul = pl.pallas_call(
    _mm_body,
    grid=(GRID,),
    in_specs=[
        pl.BlockSpec((RB, D), lambda i: (i, 0)),
        pl.BlockSpec((D, D), lambda i: (0, 0)),
    ],
    out_specs=pl.BlockSpec((RB, D), lambda i: (i, 0)),
    out_shape=jax.ShapeDtypeStruct((NP, D), jnp.float32),
)


def _lin1_body(xw_ref, d0_ref, d1_ref, y_ref, dinv_ref):
    deg = d0_ref[...] + d1_ref[...] + 1.0
    dv = jax.lax.rsqrt(deg)
    dinv_ref[...] = dv
    y_ref[...] = xw_ref[...] * dv


_tc_lin1 = pl.pallas_call(
    _lin1_body,
    grid=(GRID,),
    in_specs=[
        pl.BlockSpec((RB, D), lambda i: (i, 0)),
        pl.BlockSpec((RB, 1), lambda i: (i, 0)),
        pl.BlockSpec((RB, 1), lambda i: (i, 0)),
    ],
    out_specs=[
        pl.BlockSpec((RB, D), lambda i: (i, 0)),
        pl.BlockSpec((RB, 1), lambda i: (i, 0)),
    ],
    out_shape=[
        jax.ShapeDtypeStruct((NP, D), jnp.float32),
        jax.ShapeDtypeStruct((NP, 1), jnp.float32),
    ],
)


def _combine_body(p_ref, dinv_ref, b_ref, w_ref, y_ref):
    dv = dinv_ref[...]
    t = dv * (p_ref[0] + p_ref[1]) + b_ref[...]
    t = jnp.maximum(t, 0.0)
    y_ref[...] = dv * jnp.dot(t, w_ref[...], precision=_HI)


_tc_combine = pl.pallas_call(
    _combine_body,
    grid=(GRID,),
    in_specs=[
        pl.BlockSpec((NC, RB, D), lambda i: (0, i, 0)),
        pl.BlockSpec((RB, 1), lambda i: (i, 0)),
        pl.BlockSpec((1, D), lambda i: (0, 0)),
        pl.BlockSpec((D, D), lambda i: (0, 0)),
    ],
    out_specs=pl.BlockSpec((RB, D), lambda i: (i, 0)),
    out_shape=jax.ShapeDtypeStruct((NP, D), jnp.float32),
)


def _final_body(p_ref, dinv_ref, b_ref, o_ref):
    o_ref[...] = dinv_ref[...] * (p_ref[0] + p_ref[1]) + b_ref[...]


_tc_final = pl.pallas_call(
    _final_body,
    grid=(FGRID,),
    in_specs=[
        pl.BlockSpec((NC, FRB, D), lambda i: (0, i, 0)),
        pl.BlockSpec((FRB, 1), lambda i: (i, 0)),
        pl.BlockSpec((1, D), lambda i: (0, 0)),
    ],
    out_specs=pl.BlockSpec((FRB, D), lambda i: (i, 0)),
    out_shape=jax.ShapeDtypeStruct((N, D), jnp.float32),
)


def kernel(data, edge_index, W1, b1, W2, b2, W3, b3):
    src = edge_index[0]
    dst = edge_index[1]
    data_p = jnp.pad(data, ((0, NP - N), (0, 0)))
    zeros_row = jnp.zeros((TROWS, D), jnp.float32)
    zeros_deg = jnp.zeros((TROWS,), jnp.float32)
    ones_deg = jnp.ones((K,), jnp.float32)

    degp = _sc_degree(dst, zeros_deg, ones_deg)
    xw1 = _tc_matmul(data_p, W1)
    d0 = degp[:NP].reshape(NP, 1)
    d1 = degp[NP:].reshape(NP, 1)
    y1, dinv = _tc_lin1(xw1, d0, d1)

    p1 = _sc_scatter(y1, src, dst, zeros_row)
    y2 = _tc_combine(p1, dinv, b1.reshape(1, D), W2)
    p2 = _sc_scatter(y2, src, dst, zeros_row)
    y3 = _tc_combine(p2, dinv, b2.reshape(1, D), W3)
    p3 = _sc_scatter(y3, src, dst, zeros_row)
    return _tc_final(p3, dinv, b3.reshape(1, D))
